# Initial kernel scaffold; baseline (speedup 1.0000x reference)
import jax
import jax.numpy as jnp
from jax.experimental import pallas as pl


def kernel(x, gamma1, gamma2, Wa, ba, Wm1, bm1, Wm2, bm2, Wb, Wc1, Wc2, Wc3, Wmp, Wp, bp):
    raise NotImplementedError("write your pallas kernel here")


# speedup vs baseline: 3.6212x; 3.6212x over previous
import functools
import math

import jax
import jax.numpy as jnp
from jax import lax
from jax.experimental import pallas as pl
from jax.experimental.pallas import tpu as pltpu
from jax.experimental.pallas import tpu_sc as plsc

B, N, D = 4, 8192, 768
NF = 32
HID = 512
K = 4096
T = 512
HALO = 19
NTB = 9
SQRTD = math.sqrt(D)
NC, NS = 2, 16
CH = 64
F32 = jnp.float32
BF16 = jnp.bfloat16


def _sigmoid(v):
    return 1.0 / (1.0 + jnp.exp(-v))


def _gelu(v):
    return 0.5 * v * (1.0 + lax.erf(v * (1.0 / math.sqrt(2.0))))


def _stage_a_body(x_ref, wa_ref, g1_ref, ba_ref, sc_ref, sum_ref, meta_ref, s_scr):
    t = pl.program_id(1)
    xb = x_ref[0]
    x3 = (xb * xb).reshape(xb.shape[0], 6, 128)
    ss = jnp.sum(jnp.sum(x3, axis=2), axis=1, keepdims=True)
    den = jnp.maximum(jnp.sqrt(ss), 1e-12)
    xn = (xb / den * g1_ref[0][None, :]) * SQRTD
    z = jnp.dot(xn.astype(BF16), wa_ref[...],
                preferred_element_type=F32)[:, 0:1] + ba_ref[0, 0]
    s = jax.nn.sigmoid(z)
    s2 = s.reshape(16, 128)
    s_scr[pl.ds(t * 16, 16), :] = s2
    sc_ref[0, pl.ds(t * 16, 16), :] = s2
    xs = xn * s
    part = jnp.sum(xs, axis=0, keepdims=True)

    @pl.when(t == 0)
    def _():
        sum_ref[0] = part

    @pl.when(t > 0)
    def _():
        sum_ref[0] = sum_ref[0] + part

    @pl.when(t == 3)
    def _():
        bits = lax.bitcast_convert_type(s_scr[:, :], jnp.int32)

        def body(_, lohi):
            lo, hi = lohi
            mid = lo + (hi - lo + 1) // 2
            cnt = jnp.sum((bits >= mid).astype(jnp.int32))
            big = cnt >= K
            return jnp.where(big, mid, lo), jnp.where(big, hi, mid - 1)

        lo, _hi = lax.fori_loop(0, 31, body, (jnp.int32(0), jnp.int32(0x3F800000)))
        ngt = jnp.sum((bits > lo).astype(jnp.int32))
        lane = lax.broadcasted_iota(jnp.int32, (1, 128), 1)
        meta_ref[0] = jnp.where(lane == 0, lo, jnp.where(lane == 1, ngt, 0))


def _stage_a(x, wa_g, g1, ba_pad):
    return pl.pallas_call(
        _stage_a_body,
        grid=(B, 4),
        in_specs=[
            pl.BlockSpec((1, N // 4, D), lambda b, t: (b, t, 0)),
            pl.BlockSpec((D, 128), lambda b, t: (0, 0)),
            pl.BlockSpec((1, D), lambda b, t: (0, 0)),
            pl.BlockSpec((1, 128), lambda b, t: (0, 0)),
        ],
        out_specs=[
            pl.BlockSpec((1, 64, 128), lambda b, t: (b, 0, 0)),
            pl.BlockSpec((1, 1, D), lambda b, t: (b, 0, 0)),
            pl.BlockSpec((1, 1, 128), lambda b, t: (b, 0, 0)),
        ],
        out_shape=[
            jax.ShapeDtypeStruct((B, 64, 128), F32),
            jax.ShapeDtypeStruct((B, 1, D), F32),
            jax.ShapeDtypeStruct((B, 1, 128), jnp.int32),
        ],
        scratch_shapes=[pltpu.VMEM((64, 128), F32)],
    )(x, wa_g, g1, ba_pad)


def _sc_select_gather(x2d, scores_flat, meta_flat):
    mesh = plsc.VectorSubcoreMesh(core_axis_name="c", subcore_axis_name="s")

    @functools.partial(
        pl.kernel,
        out_type=jax.ShapeDtypeStruct((B * K, D), F32),
        mesh=mesh,
        scratch_types=[
            pltpu.VMEM((N,), F32),
            pltpu.VMEM((K,), jnp.int32),
            pltpu.VMEM((16,), jnp.int32),
            pltpu.VMEM((CH,), jnp.int32),
            pltpu.VMEM((CH, D), F32),
            pltpu.VMEM_SHARED((2, K), jnp.int32),
            pltpu.SemaphoreType.DMA,
        ],
        compiler_params=pltpu.CompilerParams(needs_layout_passes=False),
    )
    def sck(x_hbm, sc_hbm, meta_hbm, out_hbm, sc_v, idx_v, meta_v, gidx_v, rows_v, sh_idx, sem):
        c = lax.axis_index("c")
        s = lax.axis_index("s")

        @pl.when(s < 2)
        def _():
            b = c + 2 * s
            pltpu.sync_copy(sc_hbm.at[pl.ds(b * N, N)], sc_v)
            pltpu.sync_copy(meta_hbm.at[pl.ds(b * 128, 16)], meta_v)
            mv = meta_v[...]
            lane = lax.broadcasted_iota(jnp.int32, (16,), 0)
            thr = jnp.sum(jnp.where(lane == 0, mv, 0))
            rem = K - jnp.sum(jnp.where(lane == 1, mv, 0))
            base = b * N

            def body(i, carry):
                off, eqs = carry
                v = sc_v[pl.ds(i * 16, 16)]
                bits = plsc.bitcast(v, jnp.int32)
                gt = bits > thr
                eqm = bits == thr
                eqi = eqm.astype(jnp.int32)
                ecs = plsc.cumsum(eqi)
                keep_eq = eqm & ((eqs + (ecs - eqi)) < rem)
                kept = gt | keep_eq
                ki = kept.astype(jnp.int32)
                kcs = plsc.cumsum(ki)
                pos = (kcs - ki) + off
                gidx = base + i * 16 + lane
                plsc.store_scatter(idx_v, [pos], gidx, mask=kept)
                return off + jnp.max(kcs), eqs + jnp.max(ecs)

            lax.fori_loop(0, N // 16, body, (jnp.int32(0), jnp.int32(0)))
            pltpu.sync_copy(idx_v, sh_idx.at[s])

        plsc.subcore_barrier()

        b_loc = s // 8
        r0 = (s % 8) * (K // 8)
        b = c + 2 * b_loc

        def gbody(k, carry):
            o = r0 + k * CH
            pltpu.sync_copy(sh_idx.at[b_loc, pl.ds(o, CH)], gidx_v)
            pltpu.async_copy(x_hbm.at[gidx_v], rows_v, sem).wait()
            pltpu.sync_copy(rows_v, out_hbm.at[pl.ds(b * K + o, CH)])
            return carry

        lax.fori_loop(0, (K // 8) // CH, gbody, 0)

    return sck(x2d, scores_flat, meta_flat)


def _stage_b_body(prev_ref, cur_ref, next_ref, sum_ref,
                  wa_ref, g1_ref, g2_ref, ba_ref,
                  wm1_ref, bm1_ref, wm2_ref, bm2_ref,
                  wb_ref, wconv_ref, wmp_ref, wp_ref, bp_ref,
                  out_ref, acc_ref):
    t = pl.program_id(1)
    win = jnp.concatenate([prev_ref[0][T - HALO:], cur_ref[0], next_ref[0][:HALO]],
                          axis=0)
    W = T + 2 * HALO
    rowg = t * T + lax.broadcasted_iota(jnp.int32, (W, 1), 0) - HALO
    ssw = jnp.sum(win * win, axis=1, keepdims=True)
    invw = SQRTD / jnp.maximum(jnp.sqrt(ssw), 1e-12)
    zw = jnp.sum(win * wa_ref[0][None, :], axis=1, keepdims=True) * invw + ba_ref[0, 0]
    sw = _sigmoid(zw)
    xsw = win * (g1_ref[0][None, :] * (invw * sw))
    valid = (rowg >= 0) & (rowg < K)
    xsw = jnp.where(valid, xsw, 0.0)
    pos = lax.broadcasted_iota(jnp.int32, (W, 1), 0)
    curmask = valid & (pos >= HALO) & (pos < HALO + T)
    tile_sum = jnp.sum(jnp.where(curmask, xsw, 0.0), axis=0, keepdims=True)
    acc = jnp.where(t == 0, 0.0, acc_ref[0][None, :]) + tile_sum
    acc_ref[0] = acc[0]
    extra = sum_ref[0] - acc
    xcw = jnp.where(rowg == K, extra, xsw)
    ss2 = jnp.sum(xcw * xcw, axis=1, keepdims=True)
    inv2 = SQRTD / jnp.maximum(jnp.sqrt(ss2), 1e-12)
    xn2 = xcw * inv2 * g2_ref[0][None, :]
    xn2c = xn2[HALO:HALO + T]
    h = _gelu(jnp.dot(xn2c.astype(BF16), wm1_ref[...],
                      preferred_element_type=F32) + bm1_ref[0][None, :])
    moe = jnp.dot(h.astype(BF16), wm2_ref[...],
                  preferred_element_type=F32) + bm2_ref[0][None, :]
    xb = jnp.dot(xn2.astype(BF16), wb_ref[...], preferred_element_type=F32)
    xw = jnp.concatenate([xb[k:k + T] for k in range(2 * HALO + 1)], axis=1)
    convs = jnp.dot(xw.astype(BF16), wconv_ref[...], preferred_element_type=F32)
    big = jnp.where((rowg >= 0) & (rowg <= K), xn2, -1e30)
    mp = jnp.maximum(jnp.maximum(big[HALO - 1:HALO - 1 + T], big[HALO:HALO + T]),
                     big[HALO + 1:HALO + 1 + T])
    xmp = jnp.dot(mp.astype(BF16), wmp_ref[...], preferred_element_type=F32)
    outs = jnp.concatenate([convs, xmp], axis=1)
    outs = _gelu(outs * (1.0 / math.sqrt(1.0 + 1e-5)))
    incep = jnp.dot(outs.astype(BF16), wp_ref[...],
                    preferred_element_type=F32) + bp_ref[0][None, :]
    y = xcw[HALO:HALO + T] + moe + incep
    out_ref[0] = _gelu(y)


def _stage_b(xg, sumall, wa_g, g1, g2, ba_pad, Wm1b, bm1r, Wm2b, bm2r,
             Wb_mat, Wconv, Wmp_mat, Wpb, bpr):
    def xg_spec(fn):
        return pl.BlockSpec((1, T, D), lambda b, t: (b, fn(t), 0))

    cl = lambda v: jnp.clip(v, 0, K // T - 1)
    full = lambda shape: pl.BlockSpec(shape, lambda b, t: tuple(0 for _ in shape))
    return pl.pallas_call(
        _stage_b_body,
        grid=(B, NTB),
        in_specs=[
            xg_spec(lambda t: cl(t - 1)),
            xg_spec(lambda t: cl(t)),
            xg_spec(lambda t: cl(t + 1)),
            pl.BlockSpec((1, 1, D), lambda b, t: (b, 0, 0)),
            full((1, D)), full((1, D)), full((1, D)), full((1, 128)),
            full((D, HID)), full((1, HID)), full((HID, D)), full((1, D)),
            full((D, NF)), full((39 * NF, 3 * NF)), full((D, NF)),
            full((4 * NF, D)), full((1, D)),
        ],
        out_specs=pl.BlockSpec((1, T, D), lambda b, t: (b, t, 0)),
        out_shape=jax.ShapeDtypeStruct((B, K + 1, D), F32),
        scratch_shapes=[pltpu.VMEM((1, D), F32)],
    )(xg, xg, xg, sumall, wa_g, g1, g2, ba_pad, Wm1b, bm1r, Wm2b, bm2r,
      Wb_mat, Wconv, Wmp_mat, Wpb, bpr)


def kernel(x, gamma1, gamma2, Wa, ba, Wm1, bm1, Wm2, bm2, Wb, Wc1, Wc2, Wc3, Wmp, Wp, bp):
    wa_g = (gamma1 * Wa[:, 0]).reshape(1, D)
    g1 = gamma1.reshape(1, D)
    g2 = gamma2.reshape(1, D)
    ba_pad = jnp.zeros((1, 128), F32).at[0, 0].set(ba[0])
    wa_bf = jnp.zeros((D, 128), BF16).at[:, 0].set(Wa[:, 0].astype(BF16))

    scores, sumall, meta = _stage_a(x, wa_bf, g1, ba_pad)

    xg2d = _sc_select_gather(x.reshape(B * N, D), scores.reshape(B * N),
                             meta.reshape(B * 128))

    Wm1b = Wm1.astype(BF16)
    Wm2b = Wm2.astype(BF16)
    Wb_mat = Wb[:, :, 0].T.astype(BF16)
    Wmp_mat = Wmp[:, :, 0].T.astype(BF16)
    Wconv = jnp.zeros((39 * NF, 3 * NF), F32)
    Wconv = Wconv.at[:, 0:NF].set(jnp.transpose(Wc1, (2, 1, 0)).reshape(39 * NF, NF))
    Wconv = Wconv.at[10 * NF:29 * NF, NF:2 * NF].set(
        jnp.transpose(Wc2, (2, 1, 0)).reshape(19 * NF, NF))
    Wconv = Wconv.at[15 * NF:24 * NF, 2 * NF:3 * NF].set(
        jnp.transpose(Wc3, (2, 1, 0)).reshape(9 * NF, NF))
    Wconv = Wconv.astype(BF16)
    Wpb = Wp.astype(BF16)

    out = _stage_b(xg2d.reshape(B, K, D), sumall, wa_g, g1, g2, ba_pad,
                   Wm1b, bm1.reshape(1, HID), Wm2b, bm2.reshape(1, D),
                   Wb_mat, Wconv, Wmp_mat, Wpb, bp.reshape(1, D))
    return out
